# baseline (speedup 1.0000x reference)
import jax
import jax.numpy as jnp
from jax import lax
from jax.experimental import pallas as pl
from jax.experimental.pallas import tpu as pltpu
from jax.experimental.pallas import tpu_sc as plsc

NC = 2
NS = 16
NW = NC * NS

V = 1000000
EMB = 32
B = 16384
L = 200

NBR = 248
NBLK = 126
VPW = NBR * NBLK
VTAIL = V - NW * VPW

BPW = B // NW
CB = 4
NCHUNK = BPW // CB
LANES = 16


def _compact_body(x_hbm, out_hbm, tile0, tile1, out0, out1,
                  isem0, isem1, osem0, osem1):
    wid = lax.axis_index("s") * NC + lax.axis_index("c")
    r0 = pl.multiple_of(wid * VPW, 8)

    tiles = (tile0, tile1)
    outs = (out0, out1)
    isems = (isem0, isem1)
    osems = (osem0, osem1)

    def src(i, n=NBR):
        return x_hbm.at[pl.ds(pl.multiple_of(r0 + i * NBR, 8), n), :]

    def compact_rows(tile_v, out_v, nrows):
        def row(r, c):
            out_v[pl.ds(r * EMB, LANES)] = tile_v[r, pl.ds(0, LANES)]
            out_v[pl.ds(r * EMB + LANES, LANES)] = tile_v[r, pl.ds(LANES, LANES)]
            return c

        lax.fori_loop(0, nrows, row, 0, unroll=8)

    pltpu.async_copy(src(0), tile0, isem0)

    def blk(i2, carry):
        for b in range(2):
            i = i2 + b
            tile_v, out_v = tiles[b], outs[b]
            pltpu.make_async_copy(src(i), tile_v, isems[b]).wait()

            @pl.when(i + 1 < NBLK)
            def _():
                pltpu.async_copy(src(i + 1), tiles[1 - b], isems[1 - b])

            @pl.when(i >= 2)
            def _():
                pltpu.make_async_copy(
                    out_v,
                    out_hbm.at[pl.ds((r0 + (i - 2) * NBR) * EMB, NBR * EMB)],
                    osems[b]).wait()

            compact_rows(tile_v, out_v, NBR)
            pltpu.async_copy(
                out_v, out_hbm.at[pl.ds((r0 + i * NBR) * EMB, NBR * EMB)],
                osems[b])
        return carry

    lax.fori_loop(0, NBLK // 2, lambda i, u: blk(i * 2, u), 0)
    for b in (0, 1):
        pltpu.make_async_copy(
            outs[b],
            out_hbm.at[pl.ds((r0 + (NBLK - 2 + b) * NBR) * EMB, NBR * EMB)],
            osems[b]).wait()

    @pl.when(wid == 0)
    def _():
        t0 = pl.multiple_of(NW * VPW, 8)
        pltpu.sync_copy(x_hbm.at[pl.ds(t0, VTAIL), :],
                        tile0.at[pl.ds(0, VTAIL), :])
        compact_rows(tile0, out0, VTAIL)
        pltpu.sync_copy(out0.at[pl.ds(0, VTAIL * EMB)],
                        out_hbm.at[pl.ds(t0 * EMB, VTAIL * EMB)])


def _lookup_body(ids_hbm, table_hbm, out_hbm,
                 idx0, idx1, rows0, rows1, out_v,
                 gsem0, gsem1, isem0, isem1):
    wid = lax.axis_index("s") * NC + lax.axis_index("c")
    base = wid * BPW
    scale = jnp.float32(1.0 / L)
    z = jnp.zeros((LANES,), jnp.float32)

    def idx_start(c):
        return (base + c * CB) * L

    pltpu.sync_copy(ids_hbm.at[pl.ds(idx_start(0), CB * L)], idx0)
    pltpu.async_copy(table_hbm.at[idx0], rows0, gsem0)
    pltpu.async_copy(ids_hbm.at[pl.ds(idx_start(1), CB * L)], idx1, isem1)

    bufs = ((idx0, rows0, gsem0), (idx1, rows1, gsem1))
    isems = (isem0, isem1)

    def outer(c2, carry):
        for b in range(2):
            c = c2 + b
            idx_c, rows_c, gsem_c = bufs[b]
            idx_n, rows_n, gsem_n = bufs[1 - b]
            pltpu.make_async_copy(table_hbm.at[idx_c], rows_c, gsem_c).wait()

            @pl.when(c + 1 < NCHUNK)
            def _():
                pltpu.make_async_copy(
                    ids_hbm.at[pl.ds(idx_start(c + 1), CB * L)],
                    idx_n, isems[1 - b]).wait()
                pltpu.async_copy(table_hbm.at[idx_n], rows_n, gsem_n)

            @pl.when(c + 2 < NCHUNK)
            def _():
                pltpu.async_copy(
                    ids_hbm.at[pl.ds(idx_start(c + 2), CB * L)],
                    idx_c, isems[b])

            for j in range(CB):
                off = j * L

                def red(i, acc):
                    a0, a1 = acc
                    return (a0 + rows_c[off + i, pl.ds(0, LANES)],
                            a1 + rows_c[off + i, pl.ds(LANES, LANES)])

                a0, a1 = lax.fori_loop(0, L, red, (z, z), unroll=8)
                row = c * CB + j
                out_v[row, pl.ds(0, LANES)] = a0 * scale
                out_v[row, pl.ds(LANES, LANES)] = a1 * scale
        return carry

    lax.fori_loop(0, NCHUNK // 2, lambda i, u: outer(i * 2, u), 0)
    pltpu.sync_copy(out_v, out_hbm.at[pl.ds(base, BPW)])


_MESH = dict(core_axis_name="c", subcore_axis_name="s",
             num_cores=NC, num_subcores=NS)


@jax.jit
def kernel(token_ids, emb_weight):
    compact = pl.kernel(
        _compact_body,
        out_type=jax.ShapeDtypeStruct((V * EMB,), jnp.float32),
        mesh=plsc.VectorSubcoreMesh(**_MESH),
        scratch_types=[
            pltpu.VMEM((NBR, EMB), jnp.float32),
            pltpu.VMEM((NBR, EMB), jnp.float32),
            pltpu.VMEM((NBR * EMB,), jnp.float32),
            pltpu.VMEM((NBR * EMB,), jnp.float32),
            pltpu.SemaphoreType.DMA,
            pltpu.SemaphoreType.DMA,
            pltpu.SemaphoreType.DMA,
            pltpu.SemaphoreType.DMA,
        ],
        compiler_params=pltpu.CompilerParams(use_tc_tiling_on_sc=True),
    )
    lookup = pl.kernel(
        _lookup_body,
        out_type=jax.ShapeDtypeStruct((B, EMB), jnp.float32),
        mesh=plsc.VectorSubcoreMesh(**_MESH),
        scratch_types=[
            pltpu.VMEM((CB * L,), jnp.int32),
            pltpu.VMEM((CB * L,), jnp.int32),
            pltpu.VMEM((CB * L, EMB), jnp.float32),
            pltpu.VMEM((CB * L, EMB), jnp.float32),
            pltpu.VMEM((BPW, EMB), jnp.float32),
            pltpu.SemaphoreType.DMA,
            pltpu.SemaphoreType.DMA,
            pltpu.SemaphoreType.DMA,
            pltpu.SemaphoreType.DMA,
        ],
        compiler_params=pltpu.CompilerParams(use_tc_tiling_on_sc=False),
    )
    table = compact(emb_weight).reshape(V, EMB)
    ids_flat = token_ids.reshape(B * L).astype(jnp.int32)
    return lookup(ids_flat, table)
